# baseline (speedup 1.0000x reference)
import jax
import jax.numpy as jnp
from jax.experimental import pallas as pl
from jax.experimental.pallas import tpu as pltpu

_BM = 400


def _rgcn_body(adj_ref, x_ref, w_ref, b_ref, o_ref):
    rel = pl.program_id(1)
    nrel = pl.num_programs(1)
    msg = jax.lax.dot(adj_ref[0], x_ref[...],
                      preferred_element_type=jnp.float32)
    part = jax.lax.dot(msg, w_ref[rel], preferred_element_type=jnp.float32)

    @pl.when(rel == 0)
    def _first():
        o_ref[...] = part + b_ref[...]

    @pl.when(rel == nrel - 1)
    def _last():
        o_ref[...] = jnp.maximum(o_ref[...] + part, 0.0)


def kernel(node_features, adj_list, weight, bias):
    n, in_dim = node_features.shape
    r = adj_list.shape[0]
    out_dim = weight.shape[-1]
    num_m = n // _BM

    b2 = bias.reshape(1, out_dim)

    return pl.pallas_call(
        _rgcn_body,
        grid=(num_m, r),
        in_specs=[
            pl.BlockSpec((1, _BM, n), lambda m, rel: (rel, m, 0)),
            pl.BlockSpec((n, in_dim), lambda m, rel: (0, 0)),
            pl.BlockSpec((r, in_dim, out_dim), lambda m, rel: (0, 0, 0)),
            pl.BlockSpec((1, out_dim), lambda m, rel: (0, 0)),
        ],
        out_specs=pl.BlockSpec((_BM, out_dim), lambda m, rel: (m, 0)),
        out_shape=jax.ShapeDtypeStruct((n, out_dim), jnp.float32),
        compiler_params=pltpu.CompilerParams(
            dimension_semantics=("arbitrary", "arbitrary"),
        ),
    )(adj_list, node_features, weight, b2)
